# baseline (speedup 1.0000x reference)
import functools

import jax
import jax.numpy as jnp
from jax import lax
from jax.experimental import pallas as pl
from jax.experimental.pallas import tpu as pltpu
from jax.experimental.pallas import tpu_sc as plsc

B = 16384
BAG = 50
K = 20
D = 64
VOCAB = 1000001
NW = 32
BPW = B // NW
SUB = 80

CA = 16
NCA = BPW // CA
ZROWS = CA * BAG
NZS = ZROWS // SUB

CB = 16
NCB = BPW // CB
VROWS = CB * K
NVS = VROWS // SUB


def _zphase(interval_flat, wz):
  mesh = plsc.VectorSubcoreMesh(core_axis_name="c", subcore_axis_name="s")

  @functools.partial(
      pl.kernel,
      out_type=jax.ShapeDtypeStruct((B * D,), jnp.float32),
      mesh=mesh,
      scratch_types=[
          pltpu.VMEM((ZROWS,), jnp.int32),
          pltpu.VMEM((ZROWS, D), jnp.float32),
          pltpu.VMEM((CA * D,), jnp.float32),
          pltpu.SemaphoreType.DMA,
      ],
      compiler_params=pltpu.CompilerParams(
          needs_layout_passes=False, use_tc_tiling_on_sc=False),
  )
  def body(interval_hbm, wz_hbm, z_hbm, iidx_v, zrows_v, zbuf_v, sem):
    wid = lax.axis_index("s") * 2 + lax.axis_index("c")
    bag0 = wid * BPW

    def chunk_body(ci, carry):
      cbag = bag0 + ci * CA
      pltpu.sync_copy(interval_hbm.at[pl.ds(cbag * BAG, ZROWS)], iidx_v)
      copies = []
      for j in range(NZS):
        copies.append(pltpu.async_copy(
            wz_hbm.at[iidx_v.at[pl.ds(j * SUB, SUB)]],
            zrows_v.at[pl.ds(j * SUB, SUB)], sem))
      for cp in copies:
        cp.wait()

      def bag_body(b, carry2):
        zbase = b * BAG

        def row_body(r, acc):
          a0, a1, a2, a3 = acc
          row = zbase + r * 2
          a0 = a0 + zrows_v[row, pl.ds(0, 16)] + zrows_v[row + 1, pl.ds(0, 16)]
          a1 = a1 + zrows_v[row, pl.ds(16, 16)] + zrows_v[row + 1, pl.ds(16, 16)]
          a2 = a2 + zrows_v[row, pl.ds(32, 16)] + zrows_v[row + 1, pl.ds(32, 16)]
          a3 = a3 + zrows_v[row, pl.ds(48, 16)] + zrows_v[row + 1, pl.ds(48, 16)]
          return (a0, a1, a2, a3)

        zero = jnp.zeros((16,), jnp.float32)
        s0, s1, s2, s3 = lax.fori_loop(0, BAG // 2, row_body,
                                       (zero, zero, zero, zero))
        scale = jnp.float32(1.0 / BAG)
        zbuf_v[pl.ds(b * D, 16)] = s0 * scale
        zbuf_v[pl.ds(b * D + 16, 16)] = s1 * scale
        zbuf_v[pl.ds(b * D + 32, 16)] = s2 * scale
        zbuf_v[pl.ds(b * D + 48, 16)] = s3 * scale
        return carry2

      lax.fori_loop(0, CA, bag_body, 0)
      pltpu.sync_copy(zbuf_v, z_hbm.at[pl.ds(cbag * D, CA * D)])
      return carry

    lax.fori_loop(0, NCA, chunk_body, 0)

  return body(interval_flat, wz)


def _dotphase(target_flat, wv_pairs, z_flat):
  mesh = plsc.VectorSubcoreMesh(core_axis_name="c", subcore_axis_name="s")

  @functools.partial(
      pl.kernel,
      out_type=jax.ShapeDtypeStruct((B * K,), jnp.float32),
      mesh=mesh,
      scratch_types=[
          pltpu.VMEM((VROWS + 16,), jnp.int32),
          pltpu.VMEM((VROWS,), jnp.int32),
          pltpu.VMEM((VROWS, 128), jnp.float32),
          pltpu.VMEM((CB * D,), jnp.float32),
          pltpu.VMEM((VROWS,), jnp.float32),
          pltpu.VMEM((VROWS * 16,), jnp.float32),
          pltpu.SemaphoreType.DMA,
      ],
      compiler_params=pltpu.CompilerParams(
          needs_layout_passes=False, use_tc_tiling_on_sc=True),
  )
  def body(target_hbm, wv_hbm, z_hbm, out_hbm,
           tidx_v, rowidx_v, vrows_v, zbuf_v, y_v, psum_v, sem):
    wid = lax.axis_index("s") * 2 + lax.axis_index("c")
    bag0 = wid * BPW

    def chunk_body(ci, carry):
      cbag = bag0 + ci * CB
      pltpu.sync_copy(target_hbm.at[pl.ds(cbag * K, VROWS)],
                      tidx_v.at[pl.ds(0, VROWS)])
      pltpu.sync_copy(z_hbm.at[pl.ds(cbag * D, CB * D)], zbuf_v)

      def shift_body(i, carry2):
        rowidx_v[pl.ds(i * 16, 16)] = lax.shift_right_logical(
            tidx_v[pl.ds(i * 16, 16)], 1)
        return carry2

      lax.fori_loop(0, VROWS // 16, shift_body, 0)

      copies = []
      for j in range(NVS):
        copies.append(pltpu.async_copy(
            wv_hbm.at[rowidx_v.at[pl.ds(j * SUB, SUB)]],
            vrows_v.at[pl.ds(j * SUB, SUB)], sem))
      for cp in copies:
        cp.wait()

      def bag_body(b, carry2):
        z0 = zbuf_v[pl.ds(b * D, 16)]
        z1 = zbuf_v[pl.ds(b * D + 16, 16)]
        z2 = zbuf_v[pl.ds(b * D + 32, 16)]
        z3 = zbuf_v[pl.ds(b * D + 48, 16)]
        vbase = b * K

        def k_body(k, carry3):
          row = vbase + k
          half = (tidx_v[pl.ds(row, 16)][0] & 1) * D
          p = z0 * vrows_v[row, pl.ds(half, 16)]
          p = p + z1 * vrows_v[row, pl.ds(half + 16, 16)]
          p = p + z2 * vrows_v[row, pl.ds(half + 32, 16)]
          p = p + z3 * vrows_v[row, pl.ds(half + 48, 16)]
          psum_v[pl.ds(row * 16, 16)] = p
          return carry3

        lax.fori_loop(0, K, k_body, 0)
        return carry2

      lax.fori_loop(0, CB, bag_body, 0)

      lanes = lax.iota(jnp.int32, 16)

      def red_body(g, carry2):
        base = (g * 16 + lanes) * 16

        def d_body(d, acc):
          return acc + plsc.load_gather(psum_v, [base + d])

        acc = lax.fori_loop(0, 16, d_body, jnp.zeros((16,), jnp.float32))
        y_v[pl.ds(g * 16, 16)] = acc
        return carry2

      lax.fori_loop(0, VROWS // 16, red_body, 0)
      pltpu.sync_copy(y_v, out_hbm.at[pl.ds(cbag * K, VROWS)])
      return carry

    lax.fori_loop(0, NCB, chunk_body, 0)

  return body(target_flat, wv_pairs, z_flat)


def kernel(interval, target, Wz, Wv):
  wv_main = Wv[: VOCAB - 1].reshape((VOCAB - 1) // 2, 2 * D)
  wv_last = jnp.concatenate(
      [Wv[VOCAB - 1:], jnp.zeros((1, D), jnp.float32)], axis=1)
  wv_pairs = jnp.concatenate([wv_main, wv_last], axis=0)

  z_flat = _zphase(interval.reshape(-1), Wz)
  return _dotphase(target.reshape(-1), wv_pairs, z_flat)


# speedup vs baseline: 1.5923x; 1.5923x over previous
import functools

import jax
import jax.numpy as jnp
from jax import lax
from jax.experimental import pallas as pl
from jax.experimental.pallas import tpu as pltpu
from jax.experimental.pallas import tpu_sc as plsc

B = 16384
BAG = 50
K = 20
D = 64
VOCAB = 1000001
NW = 32
BPW = B // NW
SUB = 80

CA = 16
NCA = BPW // CA
ZROWS = CA * BAG
NZS = ZROWS // SUB

CB = 16
NCB = BPW // CB
VROWS = CB * K
NVS = VROWS // SUB

PAIR_BLK = 2048
NPB = 245
HALF_V = NPB * PAIR_BLK


def _pack_pairs(wvt):
  def body(lo_ref, hi_ref, o_ref):
    o_ref[:, pl.ds(0, D)] = lo_ref[...].T
    o_ref[:, pl.ds(D, D)] = hi_ref[...].T

  return pl.pallas_call(
      body,
      grid=(NPB,),
      in_specs=[
          pl.BlockSpec((D, PAIR_BLK), lambda i: (0, i)),
          pl.BlockSpec(
              (D, PAIR_BLK),
              lambda i: (0, jnp.minimum(i + NPB, VOCAB // PAIR_BLK))),
      ],
      out_specs=pl.BlockSpec((PAIR_BLK, 2 * D), lambda i: (i, 0)),
      out_shape=jax.ShapeDtypeStruct((HALF_V, 2 * D), jnp.float32),
  )(wvt, wvt)


def _zphase(interval_flat, wz):
  mesh = plsc.VectorSubcoreMesh(core_axis_name="c", subcore_axis_name="s")

  @functools.partial(
      pl.kernel,
      out_type=jax.ShapeDtypeStruct((B * D,), jnp.float32),
      mesh=mesh,
      scratch_types=[
          pltpu.VMEM((ZROWS,), jnp.int32),
          pltpu.VMEM((ZROWS,), jnp.int32),
          pltpu.VMEM((ZROWS, D), jnp.float32),
          pltpu.VMEM((ZROWS, D), jnp.float32),
          pltpu.VMEM((CA * D,), jnp.float32),
          pltpu.SemaphoreType.DMA,
          pltpu.SemaphoreType.DMA,
      ],
      compiler_params=pltpu.CompilerParams(
          needs_layout_passes=False, use_tc_tiling_on_sc=False),
  )
  def body(interval_hbm, wz_hbm, z_hbm, iidx0_v, iidx1_v, zrows0_v, zrows1_v,
           zbuf_v, sem0, sem1):
    wid = lax.axis_index("s") * 2 + lax.axis_index("c")
    bag0 = wid * BPW
    sems = (sem0, sem1)
    iidxs = (iidx0_v, iidx1_v)
    zrowss = (zrows0_v, zrows1_v)

    def fire(ci, buf):
      cbag = bag0 + ci * CA
      pltpu.sync_copy(interval_hbm.at[pl.ds(cbag * BAG, ZROWS)], iidxs[buf])
      for j in range(NZS):
        pltpu.async_copy(
            wz_hbm.at[iidxs[buf].at[pl.ds(j * SUB, SUB)]],
            zrowss[buf].at[pl.ds(j * SUB, SUB)], sems[buf])

    def drain(buf):
      for j in range(NZS):
        pltpu.make_async_copy(
            wz_hbm.at[pl.ds(0, SUB)],
            zrowss[buf].at[pl.ds(j * SUB, SUB)], sems[buf]).wait()

    def compute(ci, buf):
      cbag = bag0 + ci * CA
      rows = zrowss[buf]

      def bag_body(b, carry2):
        zbase = b * BAG

        def row_body(r, acc):
          a0, a1, a2, a3 = acc
          row = zbase + r * 2
          a0 = a0 + rows[row, pl.ds(0, 16)] + rows[row + 1, pl.ds(0, 16)]
          a1 = a1 + rows[row, pl.ds(16, 16)] + rows[row + 1, pl.ds(16, 16)]
          a2 = a2 + rows[row, pl.ds(32, 16)] + rows[row + 1, pl.ds(32, 16)]
          a3 = a3 + rows[row, pl.ds(48, 16)] + rows[row + 1, pl.ds(48, 16)]
          return (a0, a1, a2, a3)

        zero = jnp.zeros((16,), jnp.float32)
        s0, s1, s2, s3 = lax.fori_loop(0, BAG // 2, row_body,
                                       (zero, zero, zero, zero))
        scale = jnp.float32(1.0 / BAG)
        zbuf_v[pl.ds(b * D, 16)] = s0 * scale
        zbuf_v[pl.ds(b * D + 16, 16)] = s1 * scale
        zbuf_v[pl.ds(b * D + 32, 16)] = s2 * scale
        zbuf_v[pl.ds(b * D + 48, 16)] = s3 * scale
        return carry2

      lax.fori_loop(0, CA, bag_body, 0)
      pltpu.sync_copy(zbuf_v, z_hbm.at[pl.ds(cbag * D, CA * D)])

    fire(0, 0)

    def chunk_pair(cp, carry):
      ci = cp * 2
      fire(ci + 1, 1)
      drain(0)
      compute(ci, 0)

      @pl.when(cp + 1 < NCA // 2)
      def _():
        fire(ci + 2, 0)

      drain(1)
      compute(ci + 1, 1)
      return carry

    lax.fori_loop(0, NCA // 2, chunk_pair, 0)

  return body(interval_flat, wz)


def _dotphase(target_flat, wv_pairs, z_flat):
  mesh = plsc.VectorSubcoreMesh(core_axis_name="c", subcore_axis_name="s")

  @functools.partial(
      pl.kernel,
      out_type=jax.ShapeDtypeStruct((B * K,), jnp.float32),
      mesh=mesh,
      scratch_types=[
          pltpu.VMEM((VROWS + 16,), jnp.int32),
          pltpu.VMEM((VROWS + 16,), jnp.int32),
          pltpu.VMEM((VROWS,), jnp.int32),
          pltpu.VMEM((VROWS,), jnp.int32),
          pltpu.VMEM((VROWS, 128), jnp.float32),
          pltpu.VMEM((VROWS, 128), jnp.float32),
          pltpu.VMEM((CB * D,), jnp.float32),
          pltpu.VMEM((CB * D,), jnp.float32),
          pltpu.VMEM((VROWS,), jnp.float32),
          pltpu.VMEM((VROWS * 16,), jnp.float32),
          pltpu.SemaphoreType.DMA,
          pltpu.SemaphoreType.DMA,
      ],
      compiler_params=pltpu.CompilerParams(
          needs_layout_passes=False, use_tc_tiling_on_sc=True),
  )
  def body(target_hbm, wv_hbm, z_hbm, out_hbm,
           tidx0_v, tidx1_v, rowidx0_v, rowidx1_v, vrows0_v, vrows1_v,
           zbuf0_v, zbuf1_v, y_v, psum_v, sem0, sem1):
    wid = lax.axis_index("s") * 2 + lax.axis_index("c")
    bag0 = wid * BPW
    sems = (sem0, sem1)
    tidxs = (tidx0_v, tidx1_v)
    rowidxs = (rowidx0_v, rowidx1_v)
    vrowss = (vrows0_v, vrows1_v)
    zbufs = (zbuf0_v, zbuf1_v)

    def fire(ci, buf):
      cbag = bag0 + ci * CB
      tidx = tidxs[buf]
      rowidx = rowidxs[buf]
      pltpu.sync_copy(target_hbm.at[pl.ds(cbag * K, VROWS)],
                      tidx.at[pl.ds(0, VROWS)])
      pltpu.sync_copy(z_hbm.at[pl.ds(cbag * D, CB * D)], zbufs[buf])

      def shift_body(i, carry2):
        t = tidx[pl.ds(i * 16, 16)]
        rowidx[pl.ds(i * 16, 16)] = jnp.where(t >= HALF_V, t - HALF_V, t)
        return carry2

      lax.fori_loop(0, VROWS // 16, shift_body, 0)
      for j in range(NVS):
        pltpu.async_copy(
            wv_hbm.at[rowidx.at[pl.ds(j * SUB, SUB)]],
            vrowss[buf].at[pl.ds(j * SUB, SUB)], sems[buf])

    def drain(buf):
      for j in range(NVS):
        pltpu.make_async_copy(
            wv_hbm.at[pl.ds(0, SUB)],
            vrowss[buf].at[pl.ds(j * SUB, SUB)], sems[buf]).wait()

    def compute(ci, buf):
      cbag = bag0 + ci * CB
      tidx = tidxs[buf]
      vrows = vrowss[buf]
      zbuf = zbufs[buf]

      def bag_body(b, carry2):
        z0 = zbuf[pl.ds(b * D, 16)]
        z1 = zbuf[pl.ds(b * D + 16, 16)]
        z2 = zbuf[pl.ds(b * D + 32, 16)]
        z3 = zbuf[pl.ds(b * D + 48, 16)]
        vbase = b * K

        def k_body(k, carry3):
          row = vbase + k
          t0 = tidx[pl.ds(row, 16)][0]
          half = jnp.where(t0 >= HALF_V, D, 0)
          p = z0 * vrows[row, pl.ds(half, 16)]
          p = p + z1 * vrows[row, pl.ds(half + 16, 16)]
          p = p + z2 * vrows[row, pl.ds(half + 32, 16)]
          p = p + z3 * vrows[row, pl.ds(half + 48, 16)]
          psum_v[pl.ds(row * 16, 16)] = p
          return carry3

        lax.fori_loop(0, K, k_body, 0)
        return carry2

      lax.fori_loop(0, CB, bag_body, 0)

      lanes = lax.iota(jnp.int32, 16)

      def red_body(g, carry2):
        base = (g * 16 + lanes) * 16

        def d_body(d, acc):
          return acc + plsc.load_gather(psum_v, [base + d])

        acc = lax.fori_loop(0, 16, d_body, jnp.zeros((16,), jnp.float32))
        y_v[pl.ds(g * 16, 16)] = acc
        return carry2

      lax.fori_loop(0, VROWS // 16, red_body, 0)
      pltpu.sync_copy(y_v, out_hbm.at[pl.ds(cbag * K, VROWS)])

    fire(0, 0)

    def chunk_pair(cp, carry):
      ci = cp * 2
      fire(ci + 1, 1)
      drain(0)
      compute(ci, 0)

      @pl.when(cp + 1 < NCB // 2)
      def _():
        fire(ci + 2, 0)

      drain(1)
      compute(ci + 1, 1)
      return carry

    lax.fori_loop(0, NCB // 2, chunk_pair, 0)

  return body(target_flat, wv_pairs, z_flat)


def kernel(interval, target, Wz, Wv):
  wv_pairs = _pack_pairs(Wv.T)
  z_flat = _zphase(interval.reshape(-1), Wz)
  return _dotphase(target.reshape(-1), wv_pairs, z_flat)


# speedup vs baseline: 2.2699x; 1.4255x over previous
import functools

import jax
import jax.numpy as jnp
from jax import lax
from jax.experimental import pallas as pl
from jax.experimental.pallas import tpu as pltpu
from jax.experimental.pallas import tpu_sc as plsc

B = 16384
BAG = 50
K = 20
D = 64
VOCAB = 1000001
NW = 32
BPW = B // NW
SUB = 80

CA = 16
NCA = BPW // CA
ZROWS = CA * BAG
NZS = ZROWS // SUB

CB = 16
NCB = BPW // CB
VROWS = CB * K
NVS = VROWS // SUB

def _lane_splat(vec, lane):
  idx = jnp.full((16, 1), lane, jnp.int32)
  return lax.gather(
      vec, idx,
      dimension_numbers=lax.GatherDimensionNumbers(
          offset_dims=(), collapsed_slice_dims=(0,), start_index_map=(0,)),
      slice_sizes=(1,),
      mode=lax.GatherScatterMode.PROMISE_IN_BOUNDS)


PAIR_BLK = 4096
NPB = 123
HALF_V = NPB * PAIR_BLK


def _pack_pairs(wvt):
  def body(lo_ref, hi_ref, o_ref):
    o_ref[:, pl.ds(0, D)] = lo_ref[...].T
    o_ref[:, pl.ds(D, D)] = hi_ref[...].T

  return pl.pallas_call(
      body,
      grid=(NPB,),
      in_specs=[
          pl.BlockSpec((D, PAIR_BLK), lambda i: (0, i)),
          pl.BlockSpec(
              (D, PAIR_BLK),
              lambda i: (0, jnp.minimum(i + NPB, VOCAB // PAIR_BLK))),
      ],
      out_specs=pl.BlockSpec((PAIR_BLK, 2 * D), lambda i: (i, 0)),
      out_shape=jax.ShapeDtypeStruct((HALF_V, 2 * D), jnp.float32),
  )(wvt, wvt)


def _zphase(interval_flat, wz):
  mesh = plsc.VectorSubcoreMesh(core_axis_name="c", subcore_axis_name="s")

  @functools.partial(
      pl.kernel,
      out_type=jax.ShapeDtypeStruct((B * D,), jnp.float32),
      mesh=mesh,
      scratch_types=[
          pltpu.VMEM((ZROWS,), jnp.int32),
          pltpu.VMEM((ZROWS,), jnp.int32),
          pltpu.VMEM((ZROWS, D), jnp.float32),
          pltpu.VMEM((ZROWS, D), jnp.float32),
          pltpu.VMEM((CA * D,), jnp.float32),
          pltpu.SemaphoreType.DMA,
          pltpu.SemaphoreType.DMA,
      ],
      compiler_params=pltpu.CompilerParams(
          needs_layout_passes=False, use_tc_tiling_on_sc=False),
  )
  def body(interval_hbm, wz_hbm, z_hbm, iidx0_v, iidx1_v, zrows0_v, zrows1_v,
           zbuf_v, sem0, sem1):
    wid = lax.axis_index("s") * 2 + lax.axis_index("c")
    bag0 = wid * BPW
    sems = (sem0, sem1)
    iidxs = (iidx0_v, iidx1_v)
    zrowss = (zrows0_v, zrows1_v)

    def fire(ci, buf):
      cbag = bag0 + ci * CA
      pltpu.sync_copy(interval_hbm.at[pl.ds(cbag * BAG, ZROWS)], iidxs[buf])
      for j in range(NZS):
        pltpu.async_copy(
            wz_hbm.at[iidxs[buf].at[pl.ds(j * SUB, SUB)]],
            zrowss[buf].at[pl.ds(j * SUB, SUB)], sems[buf])

    def drain(buf):
      for j in range(NZS):
        pltpu.make_async_copy(
            wz_hbm.at[pl.ds(0, SUB)],
            zrowss[buf].at[pl.ds(j * SUB, SUB)], sems[buf]).wait()

    def compute(ci, buf):
      cbag = bag0 + ci * CA
      rows = zrowss[buf]

      def bag_body(b, carry2):
        zbase = b * BAG

        def row_body(r, acc):
          a0, a1, a2, a3 = acc
          row = zbase + r * 2
          a0 = a0 + rows[row, pl.ds(0, 16)] + rows[row + 1, pl.ds(0, 16)]
          a1 = a1 + rows[row, pl.ds(16, 16)] + rows[row + 1, pl.ds(16, 16)]
          a2 = a2 + rows[row, pl.ds(32, 16)] + rows[row + 1, pl.ds(32, 16)]
          a3 = a3 + rows[row, pl.ds(48, 16)] + rows[row + 1, pl.ds(48, 16)]
          return (a0, a1, a2, a3)

        zero = jnp.zeros((16,), jnp.float32)
        s0, s1, s2, s3 = lax.fori_loop(0, BAG // 2, row_body,
                                       (zero, zero, zero, zero), unroll=5)
        scale = jnp.float32(1.0 / BAG)
        zbuf_v[pl.ds(b * D, 16)] = s0 * scale
        zbuf_v[pl.ds(b * D + 16, 16)] = s1 * scale
        zbuf_v[pl.ds(b * D + 32, 16)] = s2 * scale
        zbuf_v[pl.ds(b * D + 48, 16)] = s3 * scale
        return carry2

      lax.fori_loop(0, CA, bag_body, 0)
      pltpu.sync_copy(zbuf_v, z_hbm.at[pl.ds(cbag * D, CA * D)])

    fire(0, 0)

    def chunk_pair(cp, carry):
      ci = cp * 2
      fire(ci + 1, 1)
      drain(0)
      compute(ci, 0)

      @pl.when(cp + 1 < NCA // 2)
      def _():
        fire(ci + 2, 0)

      drain(1)
      compute(ci + 1, 1)
      return carry

    lax.fori_loop(0, NCA // 2, chunk_pair, 0)

  return body(interval_flat, wz)


def _dotphase(target_flat, wv_pairs, z_flat):
  mesh = plsc.VectorSubcoreMesh(core_axis_name="c", subcore_axis_name="s")

  @functools.partial(
      pl.kernel,
      out_type=jax.ShapeDtypeStruct((B * K,), jnp.float32),
      mesh=mesh,
      scratch_types=[
          pltpu.VMEM((VROWS + 16,), jnp.int32),
          pltpu.VMEM((VROWS + 16,), jnp.int32),
          pltpu.VMEM((VROWS,), jnp.int32),
          pltpu.VMEM((VROWS,), jnp.int32),
          pltpu.VMEM((VROWS, 128), jnp.float32),
          pltpu.VMEM((VROWS, 128), jnp.float32),
          pltpu.VMEM((CB * D,), jnp.float32),
          pltpu.VMEM((CB * D,), jnp.float32),
          pltpu.VMEM((VROWS,), jnp.float32),
          pltpu.VMEM((VROWS * 16,), jnp.float32),
          pltpu.SemaphoreType.DMA,
          pltpu.SemaphoreType.DMA,
      ],
      compiler_params=pltpu.CompilerParams(
          needs_layout_passes=False, use_tc_tiling_on_sc=True),
  )
  def body(target_hbm, wv_hbm, z_hbm, out_hbm,
           tidx0_v, tidx1_v, rowidx0_v, rowidx1_v, vrows0_v, vrows1_v,
           zbuf0_v, zbuf1_v, y_v, psum_v, sem0, sem1):
    wid = lax.axis_index("s") * 2 + lax.axis_index("c")
    bag0 = wid * BPW
    sems = (sem0, sem1)
    tidxs = (tidx0_v, tidx1_v)
    rowidxs = (rowidx0_v, rowidx1_v)
    vrowss = (vrows0_v, vrows1_v)
    zbufs = (zbuf0_v, zbuf1_v)

    def fire(ci, buf):
      cbag = bag0 + ci * CB
      tidx = tidxs[buf]
      rowidx = rowidxs[buf]
      pltpu.sync_copy(target_hbm.at[pl.ds(cbag * K, VROWS)],
                      tidx.at[pl.ds(0, VROWS)])
      pltpu.sync_copy(z_hbm.at[pl.ds(cbag * D, CB * D)], zbufs[buf])

      def shift_body(i, carry2):
        t = tidx[pl.ds(i * 16, 16)]
        rowidx[pl.ds(i * 16, 16)] = jnp.where(t >= HALF_V, t - HALF_V, t)
        return carry2

      lax.fori_loop(0, VROWS // 16, shift_body, 0)
      for j in range(NVS):
        pltpu.async_copy(
            wv_hbm.at[rowidx.at[pl.ds(j * SUB, SUB)]],
            vrowss[buf].at[pl.ds(j * SUB, SUB)], sems[buf])

    def drain(buf):
      for j in range(NVS):
        pltpu.make_async_copy(
            wv_hbm.at[pl.ds(0, SUB)],
            vrowss[buf].at[pl.ds(j * SUB, SUB)], sems[buf]).wait()

    def compute(ci, buf):
      cbag = bag0 + ci * CB
      tidx = tidxs[buf]
      vrows = vrowss[buf]
      zbuf = zbufs[buf]

      def bag_body(b, carry2):
        z0 = zbuf[pl.ds(b * D, 16)]
        z1 = zbuf[pl.ds(b * D + 16, 16)]
        z2 = zbuf[pl.ds(b * D + 32, 16)]
        z3 = zbuf[pl.ds(b * D + 48, 16)]
        vbase = b * K

        tk = tidx[pl.ds(vbase, 16)]
        tk2 = tidx[pl.ds(vbase + 16, 16)]
        for k in range(K):
          row = vbase + k
          src = tk if k < 16 else tk2
          tsplat = _lane_splat(src, k % 16)
          mask = tsplat >= HALF_V
          plo = z0 * vrows[row, pl.ds(0, 16)]
          plo = plo + z1 * vrows[row, pl.ds(16, 16)]
          plo = plo + z2 * vrows[row, pl.ds(32, 16)]
          plo = plo + z3 * vrows[row, pl.ds(48, 16)]
          phi = z0 * vrows[row, pl.ds(64, 16)]
          phi = phi + z1 * vrows[row, pl.ds(80, 16)]
          phi = phi + z2 * vrows[row, pl.ds(96, 16)]
          phi = phi + z3 * vrows[row, pl.ds(112, 16)]
          psum_v[pl.ds(row * 16, 16)] = jnp.where(mask, phi, plo)
        return carry2

      lax.fori_loop(0, CB, bag_body, 0)

      lanes = lax.iota(jnp.int32, 16)

      def red_body(g, carry2):
        base = (g * 16 + lanes) * 16

        def d_body(d, acc):
          return acc + plsc.load_gather(psum_v, [base + d])

        acc = lax.fori_loop(0, 16, d_body, jnp.zeros((16,), jnp.float32),
                            unroll=4)
        y_v[pl.ds(g * 16, 16)] = acc
        return carry2

      lax.fori_loop(0, VROWS // 16, red_body, 0)
      pltpu.sync_copy(y_v, out_hbm.at[pl.ds(cbag * K, VROWS)])

    fire(0, 0)

    def chunk_pair(cp, carry):
      ci = cp * 2
      fire(ci + 1, 1)
      drain(0)
      compute(ci, 0)

      @pl.when(cp + 1 < NCB // 2)
      def _():
        fire(ci + 2, 0)

      drain(1)
      compute(ci + 1, 1)
      return carry

    lax.fori_loop(0, NCB // 2, chunk_pair, 0)

  return body(target_flat, wv_pairs, z_flat)


def kernel(interval, target, Wz, Wv):
  wv_pairs = _pack_pairs(Wv.T)
  z_flat = _zphase(interval.reshape(-1), Wz)
  return _dotphase(target.reshape(-1), wv_pairs, z_flat)
